# baseline (device time: 45662 ns/iter reference)
import jax
import jax.numpy as jnp
from jax import lax
from jax.experimental import pallas as pl
from jax.experimental.pallas import tpu as pltpu


def kernel(Q, K, V):
    b, s, h, d = Q.shape
    scale = d ** -0.5

    def body(
        q_ref,
        k_ref,
        v_ref,
        out_ref,
        k_mine,
        v_mine,
        k_theirs,
        v_theirs,
        send_sems,
        recv_sems,
    ):
        my_x = lax.axis_index("x")
        my_y = lax.axis_index("y")
        nbr = (my_x, 1 - my_y)

        barrier_sem = pltpu.get_barrier_semaphore()
        pl.semaphore_signal(
            barrier_sem, inc=1, device_id=nbr,
            device_id_type=pl.DeviceIdType.MESH,
        )
        pl.semaphore_wait(barrier_sem, 1)

        k_mine[...] = k_ref[...].astype(jnp.bfloat16)
        v_mine[...] = v_ref[...].astype(jnp.bfloat16)

        rdma_k = pltpu.make_async_remote_copy(
            src_ref=k_mine,
            dst_ref=k_theirs,
            send_sem=send_sems.at[0],
            recv_sem=recv_sems.at[0],
            device_id=nbr,
            device_id_type=pl.DeviceIdType.MESH,
        )
        rdma_v = pltpu.make_async_remote_copy(
            src_ref=v_mine,
            dst_ref=v_theirs,
            send_sem=send_sems.at[1],
            recv_sem=recv_sems.at[1],
            device_id=nbr,
            device_id_type=pl.DeviceIdType.MESH,
        )
        rdma_k.start()
        rdma_v.start()
        rdma_k.wait()
        rdma_v.wait()

        for bi in range(b):
            for hi in range(h):
                q = q_ref[bi, :, hi, :].astype(jnp.bfloat16)
                kk = jnp.concatenate(
                    [k_mine[bi, :, hi, :], k_theirs[bi, :, hi, :]], axis=0
                )
                vv = jnp.concatenate(
                    [v_mine[bi, :, hi, :], v_theirs[bi, :, hi, :]], axis=0
                )
                sc = lax.dot_general(
                    q, kk, (((1,), (1,)), ((), ())),
                    preferred_element_type=jnp.float32,
                ) * scale
                m = jnp.max(sc, axis=1, keepdims=True)
                p = jnp.exp(sc - m)
                l = jnp.sum(p, axis=1, keepdims=True)
                o = lax.dot_general(
                    p.astype(jnp.bfloat16), vv, (((1,), (0,)), ((), ())),
                    preferred_element_type=jnp.float32,
                )
                out_ref[bi, :, hi, :] = o / l

    return pl.pallas_call(
        body,
        out_shape=jax.ShapeDtypeStruct((b, s, h, d), jnp.float32),
        in_specs=[
            pl.BlockSpec(memory_space=pltpu.VMEM),
            pl.BlockSpec(memory_space=pltpu.VMEM),
            pl.BlockSpec(memory_space=pltpu.VMEM),
        ],
        out_specs=pl.BlockSpec(memory_space=pltpu.VMEM),
        scratch_shapes=[
            pltpu.VMEM((b, s, h, d), jnp.bfloat16),
            pltpu.VMEM((b, s, h, d), jnp.bfloat16),
            pltpu.VMEM((b, s, h, d), jnp.bfloat16),
            pltpu.VMEM((b, s, h, d), jnp.bfloat16),
            pltpu.SemaphoreType.DMA((2,)),
            pltpu.SemaphoreType.DMA((2,)),
        ],
        compiler_params=pltpu.CompilerParams(collective_id=0),
    )(Q, K, V)


# device time: 26216 ns/iter; 1.7418x vs baseline; 1.7418x over previous
import jax
import jax.numpy as jnp
from jax import lax
from jax.experimental import pallas as pl
from jax.experimental.pallas import tpu as pltpu


def kernel(Q, K, V):
    b, s, h, d = Q.shape
    hd = h * d
    scale = d ** -0.5

    def body(q_ref, k_ref, v_ref, out_ref, k_parts, v_parts, send_sems, recv_sems):
        my_x = lax.axis_index("x")
        my_y = lax.axis_index("y")
        nbr = (my_x, 1 - my_y)

        barrier_sem = pltpu.get_barrier_semaphore()
        pl.semaphore_signal(
            barrier_sem, inc=1, device_id=nbr,
            device_id_type=pl.DeviceIdType.MESH,
        )
        pl.semaphore_wait(barrier_sem, 1)

        k_parts[0] = k_ref[...].astype(jnp.bfloat16)
        v_parts[0] = v_ref[...].astype(jnp.bfloat16)

        rdma_k = pltpu.make_async_remote_copy(
            src_ref=k_parts.at[0],
            dst_ref=k_parts.at[1],
            send_sem=send_sems.at[0],
            recv_sem=recv_sems.at[0],
            device_id=nbr,
            device_id_type=pl.DeviceIdType.MESH,
        )
        rdma_v = pltpu.make_async_remote_copy(
            src_ref=v_parts.at[0],
            dst_ref=v_parts.at[1],
            send_sem=send_sems.at[1],
            recv_sem=recv_sems.at[1],
            device_id=nbr,
            device_id_type=pl.DeviceIdType.MESH,
        )
        rdma_k.start()
        rdma_v.start()
        rdma_k.wait()
        rdma_v.wait()

        for bi in range(b):
            for hi in range(h):
                sl = slice(hi * d, (hi + 1) * d)
                q = q_ref[bi, :, sl].astype(jnp.bfloat16)
                kk = jnp.concatenate(
                    [k_parts[0, bi, :, sl], k_parts[1, bi, :, sl]], axis=0
                )
                vv = jnp.concatenate(
                    [v_parts[0, bi, :, sl], v_parts[1, bi, :, sl]], axis=0
                )
                sc = lax.dot_general(
                    q, kk, (((1,), (1,)), ((), ())),
                    preferred_element_type=jnp.float32,
                ) * scale
                m = jnp.max(sc, axis=1, keepdims=True)
                p = jnp.exp(sc - m)
                l = jnp.sum(p, axis=1, keepdims=True)
                o = lax.dot_general(
                    p.astype(jnp.bfloat16), vv, (((1,), (0,)), ((), ())),
                    preferred_element_type=jnp.float32,
                )
                out_ref[bi, :, sl] = o / l

    out = pl.pallas_call(
        body,
        out_shape=jax.ShapeDtypeStruct((b, s, hd), jnp.float32),
        in_specs=[
            pl.BlockSpec(memory_space=pltpu.VMEM),
            pl.BlockSpec(memory_space=pltpu.VMEM),
            pl.BlockSpec(memory_space=pltpu.VMEM),
        ],
        out_specs=pl.BlockSpec(memory_space=pltpu.VMEM),
        scratch_shapes=[
            pltpu.VMEM((2, b, s, hd), jnp.bfloat16),
            pltpu.VMEM((2, b, s, hd), jnp.bfloat16),
            pltpu.SemaphoreType.DMA((2,)),
            pltpu.SemaphoreType.DMA((2,)),
        ],
        compiler_params=pltpu.CompilerParams(collective_id=0),
    )(Q.reshape(b, s, hd), K.reshape(b, s, hd), V.reshape(b, s, hd))
    return out.reshape(b, s, h, d)


# device time: 13975 ns/iter; 3.2674x vs baseline; 1.8759x over previous
import os

import jax
import jax.numpy as jnp
from jax import lax
from jax.experimental import pallas as pl
from jax.experimental.pallas import tpu as pltpu

_PROBE = os.environ.get("KERNEL_PROBE", "")


def kernel(Q, K, V):
    b, s, h, d = Q.shape
    hd = h * d
    scale = d ** -0.5

    def body(q_ref, k_ref, v_ref, out_ref, k_parts, v_parts, send_sems, recv_sems):
        my_x = lax.axis_index("x")
        my_y = lax.axis_index("y")
        nbr = (my_x, 1 - my_y)

        barrier_sem = pltpu.get_barrier_semaphore()
        pl.semaphore_signal(
            barrier_sem, inc=1, device_id=nbr,
            device_id_type=pl.DeviceIdType.MESH,
        )
        pl.semaphore_wait(barrier_sem, 1)

        k_parts[0] = k_ref[...].astype(jnp.bfloat16)
        v_parts[0] = v_ref[...].astype(jnp.bfloat16)

        if _PROBE == "compute":
            k_parts[1] = k_parts[0]
            v_parts[1] = v_parts[0]
        else:
            _exchange(k_parts, v_parts, send_sems, recv_sems, nbr)

        if _PROBE == "comm":
            out_ref[...] = q_ref[...]
            return

        for bi in range(b):
            for hi in range(h):
                sl = slice(hi * d, (hi + 1) * d)
                q = q_ref[bi, :, sl].astype(jnp.bfloat16)
                kk = jnp.concatenate(
                    [k_parts[0, bi, :, sl], k_parts[1, bi, :, sl]], axis=0
                )
                vv = jnp.concatenate(
                    [v_parts[0, bi, :, sl], v_parts[1, bi, :, sl]], axis=0
                )
                sc = lax.dot_general(
                    q, kk, (((1,), (1,)), ((), ())),
                    preferred_element_type=jnp.float32,
                ) * scale
                m = jnp.max(sc, axis=1, keepdims=True)
                p = jnp.exp(sc - m)
                l = jnp.sum(p, axis=1, keepdims=True)
                o = lax.dot_general(
                    p.astype(jnp.bfloat16), vv, (((1,), (0,)), ((), ())),
                    preferred_element_type=jnp.float32,
                )
                out_ref[bi, :, sl] = o / l

    def _exchange(k_parts, v_parts, send_sems, recv_sems, nbr):
        rdma_k = pltpu.make_async_remote_copy(
            src_ref=k_parts.at[0],
            dst_ref=k_parts.at[1],
            send_sem=send_sems.at[0],
            recv_sem=recv_sems.at[0],
            device_id=nbr,
            device_id_type=pl.DeviceIdType.MESH,
        )
        rdma_v = pltpu.make_async_remote_copy(
            src_ref=v_parts.at[0],
            dst_ref=v_parts.at[1],
            send_sem=send_sems.at[1],
            recv_sem=recv_sems.at[1],
            device_id=nbr,
            device_id_type=pl.DeviceIdType.MESH,
        )
        rdma_k.start()
        rdma_v.start()
        rdma_k.wait()
        rdma_v.wait()

    out = pl.pallas_call(
        body,
        out_shape=jax.ShapeDtypeStruct((b, s, hd), jnp.float32),
        in_specs=[
            pl.BlockSpec(memory_space=pltpu.VMEM),
            pl.BlockSpec(memory_space=pltpu.VMEM),
            pl.BlockSpec(memory_space=pltpu.VMEM),
        ],
        out_specs=pl.BlockSpec(memory_space=pltpu.VMEM),
        scratch_shapes=[
            pltpu.VMEM((2, b, s, hd), jnp.bfloat16),
            pltpu.VMEM((2, b, s, hd), jnp.bfloat16),
            pltpu.SemaphoreType.DMA((2,)),
            pltpu.SemaphoreType.DMA((2,)),
        ],
        compiler_params=pltpu.CompilerParams(collective_id=0),
    )(Q.reshape(b, s, hd), K.reshape(b, s, hd), V.reshape(b, s, hd))
    return out.reshape(b, s, h, d)
